# baseline (speedup 1.0000x reference)
import functools

import jax
import jax.numpy as jnp
from jax import lax
from jax.experimental import pallas as pl
from jax.experimental.pallas import tpu as pltpu
from jax.experimental.pallas import tpu_sc as plsc

_N = 10000
_D_IN = 256
_NSUP = 4
_D_PER = 64
_N_EDGES = 160000

_NC = 2
_NSUB = 16
_CH = 128
_NCHUNK = 80
_EPT = _CH * _NCHUNK
_EPAD = _EPT * _NSUB
_NPAD = 10240
_RPT = _NPAD // _NSUB
_OCH = 128
_DEPTH = 3


def _mm_body(x_ref, w_ref, o_ref):
    x = x_ref[...]
    for i in range(_NSUP):
        o_ref[i] = jnp.dot(x, w_ref[i], preferred_element_type=jnp.float32)


def _project(x_all, weights):
    m = x_all.shape[0]
    mb = 1000
    return pl.pallas_call(
        _mm_body,
        grid=(m // mb,),
        in_specs=[
            pl.BlockSpec((mb, _D_IN), lambda r: (r, 0)),
            pl.BlockSpec((_NSUP, _D_IN, _D_PER), lambda r: (0, 0, 0)),
        ],
        out_specs=pl.BlockSpec((_NSUP, mb, _D_PER), lambda r: (0, r, 0)),
        out_shape=jax.ShapeDtypeStruct((_NSUP, m, _D_PER), jnp.float32),
    )(x_all, weights)


def _sc_body(tab, src_hbm, dst_hbm, vals_hbm, out_hbm,
             src2, dst2, vals2, rbufs, sbufs, acc, sem_g, sem_s):
    c = lax.axis_index("c")
    s = lax.axis_index("s")
    nd = _DEPTH

    for tt in range(4):
        t = c * 4 + tt

        pltpu.sync_copy(src_hbm.at[t, s], src2)
        pltpu.sync_copy(dst_hbm.at[t, s], dst2)
        pltpu.sync_copy(vals_hbm.at[t, s], vals2)

        def _zb(r, carry):
            for k in range(4):
                sbufs[0, r, pl.ds(k * 16, 16)] = jnp.zeros((16,), jnp.float32)
            return carry
        lax.fori_loop(0, _OCH, _zb, 0)
        for k in range(_RPT // _OCH):
            pltpu.sync_copy(sbufs.at[0, pl.ds(0, _OCH)],
                            acc.at[pl.ds(s * _RPT + k * _OCH, _OCH)])
        plsc.subcore_barrier()

        def _scale(j, b):
            def _sg(g, c2):
                vv = vals2[j, pl.ds(g * 16, 16)]
                for lane in range(16):
                    e = g * 16 + lane
                    v = vv[lane]
                    for k in range(4):
                        sl = pl.ds(k * 16, 16)
                        sbufs[b, e, sl] = rbufs[b, e, sl] * v
                return c2
            lax.fori_loop(0, _CH // 16, _sg, 0)

        for j0 in range(nd):
            pltpu.async_copy(tab.at[src2.at[j0]], rbufs.at[j0], sem_g)

        def _step(j, carry):
            b = lax.rem(j, nd)
            pltpu.make_async_copy(tab.at[src2.at[j]], rbufs.at[b], sem_g).wait()

            @pl.when(j >= nd)
            def _drain():
                pltpu.make_async_copy(sbufs.at[b], acc.at[dst2.at[j]],
                                      sem_s).wait()

            _scale(j, b)

            @pl.when(j < _NCHUNK - nd)
            def _next():
                pltpu.async_copy(tab.at[src2.at[j + nd]], rbufs.at[b], sem_g)

            pltpu.async_copy(sbufs.at[b], acc.at[dst2.at[j]], sem_s, add=True)
            return carry
        lax.fori_loop(0, _NCHUNK, _step, 0)
        for j0 in range(nd):
            pltpu.make_async_copy(sbufs.at[j0], acc.at[dst2.at[j0]],
                                  sem_s).wait()
        plsc.subcore_barrier()

        for k in range(_RPT // _OCH):
            r0 = s * _RPT + k * _OCH
            pltpu.sync_copy(acc.at[pl.ds(r0, _OCH)], rbufs.at[0, pl.ds(0, _OCH)])

            def _relu(r, c3):
                for q in range(4):
                    sl = pl.ds(q * 16, 16)
                    rbufs[0, r, sl] = jnp.maximum(rbufs[0, r, sl], 0.0)
                return c3
            lax.fori_loop(0, _OCH, _relu, 0)
            pltpu.sync_copy(rbufs.at[0, pl.ds(0, _OCH)],
                            out_hbm.at[t, pl.ds(r0, _OCH)])


def _sc_scatter(tab_flat, src_all, dst_all, vals_all):
    mesh = plsc.VectorSubcoreMesh(core_axis_name="c", subcore_axis_name="s")
    f = pl.kernel(
        _sc_body,
        out_type=jax.ShapeDtypeStruct((2 * _NSUP, _NPAD, _D_PER), jnp.float32),
        mesh=mesh,
        scratch_types=[
            pltpu.VMEM((_NCHUNK, _CH), jnp.int32),
            pltpu.VMEM((_NCHUNK, _CH), jnp.int32),
            pltpu.VMEM((_NCHUNK, _CH), jnp.float32),
            pltpu.VMEM((_DEPTH, _CH, _D_PER), jnp.float32),
            pltpu.VMEM((_DEPTH, _CH, _D_PER), jnp.float32),
            pltpu.VMEM_SHARED((_NPAD, _D_PER), jnp.float32),
            pltpu.SemaphoreType.DMA,
            pltpu.SemaphoreType.DMA,
        ],
        compiler_params=pltpu.CompilerParams(use_tc_tiling_on_sc=False),
    )
    return f(tab_flat, src_all, dst_all, vals_all)


def kernel(x_u, x_v, edge_index, edge_values, weights_u):
    ei = edge_index.astype(jnp.int32)
    rows, cols = ei[:, 0], ei[:, 1]
    vals = edge_values.astype(jnp.float32)

    x_all = jnp.concatenate([x_u, x_v], axis=0)
    tab = _project(x_all, weights_u)
    tab_flat = tab.reshape(_NSUP * 2 * _N, _D_PER)

    sup = jnp.arange(_NSUP, dtype=jnp.int32)[:, None]
    src_all = jnp.stack([cols + sup * (2 * _N) + _N,
                         rows + sup * (2 * _N)], axis=1).reshape(8, _N_EDGES)
    dst_all = jnp.stack([rows, cols], axis=1).reshape(8, _N_EDGES)
    vals_all = jnp.stack([vals, vals], axis=1).reshape(8, _N_EDGES)

    pad = _EPAD - _N_EDGES
    src_all = jnp.pad(src_all, ((0, 0), (0, pad)))
    dst_all = jnp.pad(dst_all, ((0, 0), (0, pad)))
    vals_all = jnp.pad(vals_all, ((0, 0), (0, pad)))
    shape4 = (8, _NSUB, _NCHUNK, _CH)

    out8 = _sc_scatter(tab_flat,
                       src_all.reshape(shape4),
                       dst_all.reshape(shape4),
                       vals_all.reshape(shape4))[:, :_N]

    z_u = out8[0::2].transpose(1, 0, 2).reshape(_N, _NSUP * _D_PER)
    z_v = out8[1::2].transpose(1, 0, 2).reshape(_N, _NSUP * _D_PER)
    return z_u, z_v


# speedup vs baseline: 1.0320x; 1.0320x over previous
import functools

import jax
import jax.numpy as jnp
from jax import lax
from jax.experimental import pallas as pl
from jax.experimental.pallas import tpu as pltpu
from jax.experimental.pallas import tpu_sc as plsc

_N = 10000
_D_IN = 256
_NSUP = 4
_D_PER = 64
_N_EDGES = 160000

_NC = 2
_NSUB = 16
_CH = 128
_NCHUNK = 81
_EPT = _CH * _NCHUNK
_EPAD = _EPT * _NSUB
_NPAD = 10240
_RPT = _NPAD // _NSUB
_OCH = 128
_DEPTH = 3


def _mm_body(x_ref, w_ref, o_ref):
    x = x_ref[...]
    for i in range(_NSUP):
        o_ref[i] = jnp.dot(x, w_ref[i], preferred_element_type=jnp.float32)


def _project(x_all, weights):
    m = x_all.shape[0]
    mb = 1000
    return pl.pallas_call(
        _mm_body,
        grid=(m // mb,),
        in_specs=[
            pl.BlockSpec((mb, _D_IN), lambda r: (r, 0)),
            pl.BlockSpec((_NSUP, _D_IN, _D_PER), lambda r: (0, 0, 0)),
        ],
        out_specs=pl.BlockSpec((_NSUP, mb, _D_PER), lambda r: (0, r, 0)),
        out_shape=jax.ShapeDtypeStruct((_NSUP, m, _D_PER), jnp.float32),
    )(x_all, weights)


def _sc_body(tab, src_hbm, dst_hbm, vals_hbm, out_hbm,
             src2, dst2, vals2, rbufs, sbufs, acc, sem_g, sem_s):
    c = lax.axis_index("c")
    s = lax.axis_index("s")
    nd = _DEPTH

    for tt in range(4):
        t = c * 4 + tt

        pltpu.sync_copy(src_hbm.at[t, s], src2)
        pltpu.sync_copy(dst_hbm.at[t, s], dst2)
        pltpu.sync_copy(vals_hbm.at[t, s], vals2)

        def _zb(r, carry):
            for k in range(4):
                sbufs[0, r, pl.ds(k * 16, 16)] = jnp.zeros((16,), jnp.float32)
            return carry
        lax.fori_loop(0, _OCH, _zb, 0)
        for k in range(_RPT // _OCH):
            pltpu.sync_copy(sbufs.at[0, pl.ds(0, _OCH)],
                            acc.at[pl.ds(s * _RPT + k * _OCH, _OCH)])
        plsc.subcore_barrier()

        def _scale(j, b):
            def _sg(g, c2):
                vv = vals2[j, pl.ds(g * 16, 16)]
                for lane in range(16):
                    e = g * 16 + lane
                    v = vv[lane]
                    for k in range(4):
                        sl = pl.ds(k * 16, 16)
                        sbufs[b, e, sl] = rbufs[b, e, sl] * v
                return c2
            lax.fori_loop(0, _CH // 16, _sg, 0)

        for j0 in range(nd):
            pltpu.async_copy(tab.at[src2.at[j0]], rbufs.at[j0], sem_g)

        def _step(st, carry):
            for k in range(nd):
                j = nd * st + k
                pltpu.make_async_copy(tab.at[src2.at[j]], rbufs.at[k],
                                      sem_g).wait()

                @pl.when(st > 0)
                def _drain():
                    pltpu.make_async_copy(sbufs.at[k], acc.at[dst2.at[j]],
                                          sem_s).wait()

                _scale(j, k)

                @pl.when(st < _NCHUNK // nd - 1)
                def _next():
                    pltpu.async_copy(tab.at[src2.at[j + nd]], rbufs.at[k],
                                     sem_g)

                pltpu.async_copy(sbufs.at[k], acc.at[dst2.at[j]], sem_s,
                                 add=True)
            return carry
        lax.fori_loop(0, _NCHUNK // nd, _step, 0)
        for j0 in range(nd):
            pltpu.make_async_copy(sbufs.at[j0], acc.at[dst2.at[j0]],
                                  sem_s).wait()
        plsc.subcore_barrier()

        for k in range(_RPT // _OCH):
            r0 = s * _RPT + k * _OCH
            pltpu.sync_copy(acc.at[pl.ds(r0, _OCH)], rbufs.at[0, pl.ds(0, _OCH)])

            def _relu(r, c3):
                for q in range(4):
                    sl = pl.ds(q * 16, 16)
                    rbufs[0, r, sl] = jnp.maximum(rbufs[0, r, sl], 0.0)
                return c3
            lax.fori_loop(0, _OCH, _relu, 0)
            pltpu.sync_copy(rbufs.at[0, pl.ds(0, _OCH)],
                            out_hbm.at[t, pl.ds(r0, _OCH)])


def _sc_scatter(tab_flat, src_all, dst_all, vals_all):
    mesh = plsc.VectorSubcoreMesh(core_axis_name="c", subcore_axis_name="s")
    f = pl.kernel(
        _sc_body,
        out_type=jax.ShapeDtypeStruct((2 * _NSUP, _NPAD, _D_PER), jnp.float32),
        mesh=mesh,
        scratch_types=[
            pltpu.VMEM((_NCHUNK, _CH), jnp.int32),
            pltpu.VMEM((_NCHUNK, _CH), jnp.int32),
            pltpu.VMEM((_NCHUNK, _CH), jnp.float32),
            pltpu.VMEM((_DEPTH, _CH, _D_PER), jnp.float32),
            pltpu.VMEM((_DEPTH, _CH, _D_PER), jnp.float32),
            pltpu.VMEM_SHARED((_NPAD, _D_PER), jnp.float32),
            pltpu.SemaphoreType.DMA,
            pltpu.SemaphoreType.DMA,
        ],
        compiler_params=pltpu.CompilerParams(use_tc_tiling_on_sc=False),
    )
    return f(tab_flat, src_all, dst_all, vals_all)


def kernel(x_u, x_v, edge_index, edge_values, weights_u):
    ei = edge_index.astype(jnp.int32)
    rows, cols = ei[:, 0], ei[:, 1]
    vals = edge_values.astype(jnp.float32)

    x_all = jnp.concatenate([x_u, x_v], axis=0)
    tab = _project(x_all, weights_u)
    tab_flat = tab.reshape(_NSUP * 2 * _N, _D_PER)

    sup = jnp.arange(_NSUP, dtype=jnp.int32)[:, None]
    src_all = jnp.stack([cols + sup * (2 * _N) + _N,
                         rows + sup * (2 * _N)], axis=1).reshape(8, _N_EDGES)
    dst_all = jnp.stack([rows, cols], axis=1).reshape(8, _N_EDGES)
    vals_all = jnp.stack([vals, vals], axis=1).reshape(8, _N_EDGES)

    pad = _EPAD - _N_EDGES
    src_all = jnp.pad(src_all, ((0, 0), (0, pad)))
    dst_all = jnp.pad(dst_all, ((0, 0), (0, pad)))
    vals_all = jnp.pad(vals_all, ((0, 0), (0, pad)))
    shape4 = (8, _NSUB, _NCHUNK, _CH)

    out8 = _sc_scatter(tab_flat,
                       src_all.reshape(shape4),
                       dst_all.reshape(shape4),
                       vals_all.reshape(shape4))[:, :_N]

    z_u = out8[0::2].transpose(1, 0, 2).reshape(_N, _NSUP * _D_PER)
    z_v = out8[1::2].transpose(1, 0, 2).reshape(_N, _NSUP * _D_PER)
    return z_u, z_v


# speedup vs baseline: 1.0325x; 1.0005x over previous
import functools

import jax
import jax.numpy as jnp
from jax import lax
from jax.experimental import pallas as pl
from jax.experimental.pallas import tpu as pltpu
from jax.experimental.pallas import tpu_sc as plsc

_N = 10000
_D_IN = 256
_NSUP = 4
_D_PER = 64
_N_EDGES = 160000

_NC = 2
_NSUB = 16
_CH = 128
_NCHUNK = 81
_EPT = _CH * _NCHUNK
_EPAD = _EPT * _NSUB
_NPAD = 10240
_RPT = _NPAD // _NSUB
_OCH = 128
_DEPTH = 3


def _mm_body(x_ref, w_ref, o_ref):
    x = x_ref[...]
    for i in range(_NSUP):
        o_ref[i] = jnp.dot(x, w_ref[i], preferred_element_type=jnp.float32)


def _project(x_all, weights):
    m = x_all.shape[0]
    mb = 1000
    return pl.pallas_call(
        _mm_body,
        grid=(m // mb,),
        in_specs=[
            pl.BlockSpec((mb, _D_IN), lambda r: (r, 0)),
            pl.BlockSpec((_NSUP, _D_IN, _D_PER), lambda r: (0, 0, 0)),
        ],
        out_specs=pl.BlockSpec((_NSUP, mb, _D_PER), lambda r: (0, r, 0)),
        out_shape=jax.ShapeDtypeStruct((_NSUP, m, _D_PER), jnp.float32),
    )(x_all, weights)


def _sc_body(tab, src_hbm, dst_hbm, vals_hbm, out_hbm,
             src2, dst2, vals2, rb0, rb1, rb2, sb0, sb1, sb2, acc, sem_g, sem_s):
    c = lax.axis_index("c")
    s = lax.axis_index("s")
    nd = _DEPTH
    rbl = (rb0, rb1, rb2)
    sbl = (sb0, sb1, sb2)

    for tt in range(4):
        t = c * 4 + tt

        pltpu.sync_copy(src_hbm.at[t, s], src2)
        pltpu.sync_copy(dst_hbm.at[t, s], dst2)
        pltpu.sync_copy(vals_hbm.at[t, s], vals2)

        def _zb(r, carry):
            for k in range(4):
                sb0[r, pl.ds(k * 16, 16)] = jnp.zeros((16,), jnp.float32)
            return carry
        lax.fori_loop(0, _OCH, _zb, 0)
        for k in range(_RPT // _OCH):
            pltpu.sync_copy(sb0.at[pl.ds(0, _OCH)],
                            acc.at[pl.ds(s * _RPT + k * _OCH, _OCH)])
        plsc.subcore_barrier()

        def _scale(j, rb, sb):
            def _sg(g, c2):
                vv = vals2[j, pl.ds(g * 16, 16)]
                for lane in range(16):
                    e = g * 16 + lane
                    v = vv[lane]
                    for k in range(4):
                        sl = pl.ds(k * 16, 16)
                        sb[e, sl] = rb[e, sl] * v
                return c2
            lax.fori_loop(0, _CH // 16, _sg, 0)

        for j0 in range(nd):
            pltpu.async_copy(tab.at[src2.at[j0]], rbl[j0], sem_g)

        def _step(st, carry):
            for k in range(nd):
                j = nd * st + k
                rb, sb = rbl[k], sbl[k]
                pltpu.make_async_copy(tab.at[src2.at[j]], rb, sem_g).wait()

                @pl.when(st > 0)
                def _drain():
                    pltpu.make_async_copy(sb, acc.at[dst2.at[j]],
                                          sem_s).wait()

                _scale(j, rb, sb)

                @pl.when(st < _NCHUNK // nd - 1)
                def _next():
                    pltpu.async_copy(tab.at[src2.at[j + nd]], rb, sem_g)

                pltpu.async_copy(sb, acc.at[dst2.at[j]], sem_s,
                                 add=True)
            return carry
        lax.fori_loop(0, _NCHUNK // nd, _step, 0)
        for j0 in range(nd):
            pltpu.make_async_copy(sbl[j0], acc.at[dst2.at[j0]],
                                  sem_s).wait()
        plsc.subcore_barrier()

        for k in range(_RPT // _OCH):
            r0 = s * _RPT + k * _OCH
            pltpu.sync_copy(acc.at[pl.ds(r0, _OCH)], rb0.at[pl.ds(0, _OCH)])

            def _relu(r, c3):
                for q in range(4):
                    sl = pl.ds(q * 16, 16)
                    rb0[r, sl] = jnp.maximum(rb0[r, sl], 0.0)
                return c3
            lax.fori_loop(0, _OCH, _relu, 0)
            pltpu.sync_copy(rb0.at[pl.ds(0, _OCH)],
                            out_hbm.at[t, pl.ds(r0, _OCH)])


def _sc_scatter(tab_flat, src_all, dst_all, vals_all):
    mesh = plsc.VectorSubcoreMesh(core_axis_name="c", subcore_axis_name="s")
    f = pl.kernel(
        _sc_body,
        out_type=jax.ShapeDtypeStruct((2 * _NSUP, _NPAD, _D_PER), jnp.float32),
        mesh=mesh,
        scratch_types=[
            pltpu.VMEM((_NCHUNK, _CH), jnp.int32),
            pltpu.VMEM((_NCHUNK, _CH), jnp.int32),
            pltpu.VMEM((_NCHUNK, _CH), jnp.float32),
            pltpu.VMEM((_CH, _D_PER), jnp.float32),
            pltpu.VMEM((_CH, _D_PER), jnp.float32),
            pltpu.VMEM((_CH, _D_PER), jnp.float32),
            pltpu.VMEM((_CH, _D_PER), jnp.float32),
            pltpu.VMEM((_CH, _D_PER), jnp.float32),
            pltpu.VMEM((_CH, _D_PER), jnp.float32),
            pltpu.VMEM_SHARED((_NPAD, _D_PER), jnp.float32),
            pltpu.SemaphoreType.DMA,
            pltpu.SemaphoreType.DMA,
        ],
        compiler_params=pltpu.CompilerParams(use_tc_tiling_on_sc=False),
    )
    return f(tab_flat, src_all, dst_all, vals_all)


def kernel(x_u, x_v, edge_index, edge_values, weights_u):
    ei = edge_index.astype(jnp.int32)
    rows, cols = ei[:, 0], ei[:, 1]
    vals = edge_values.astype(jnp.float32)

    x_all = jnp.concatenate([x_u, x_v], axis=0)
    tab = _project(x_all, weights_u)
    tab_flat = tab.reshape(_NSUP * 2 * _N, _D_PER)

    sup = jnp.arange(_NSUP, dtype=jnp.int32)[:, None]
    src_all = jnp.stack([cols + sup * (2 * _N) + _N,
                         rows + sup * (2 * _N)], axis=1).reshape(8, _N_EDGES)
    dst_all = jnp.stack([rows, cols], axis=1).reshape(8, _N_EDGES)
    vals_all = jnp.stack([vals, vals], axis=1).reshape(8, _N_EDGES)

    pad = _EPAD - _N_EDGES
    src_all = jnp.pad(src_all, ((0, 0), (0, pad)))
    dst_all = jnp.pad(dst_all, ((0, 0), (0, pad)))
    vals_all = jnp.pad(vals_all, ((0, 0), (0, pad)))
    shape4 = (8, _NSUB, _NCHUNK, _CH)

    out8 = _sc_scatter(tab_flat,
                       src_all.reshape(shape4),
                       dst_all.reshape(shape4),
                       vals_all.reshape(shape4))[:, :_N]

    z_u = out8[0::2].transpose(1, 0, 2).reshape(_N, _NSUP * _D_PER)
    z_v = out8[1::2].transpose(1, 0, 2).reshape(_N, _NSUP * _D_PER)
    return z_u, z_v


# speedup vs baseline: 1.2931x; 1.2524x over previous
import functools

import jax
import jax.numpy as jnp
from jax import lax
from jax.experimental import pallas as pl
from jax.experimental.pallas import tpu as pltpu
from jax.experimental.pallas import tpu_sc as plsc

_N = 10000
_D_IN = 256
_NSUP = 4
_D_PER = 64
_N_EDGES = 160000

_NC = 2
_NSUB = 16
_CH = 128
_NCHUNK = 80
_EPT = _CH * _NCHUNK
_EPAD = _EPT * _NSUB
_NPAD = 10240
_RPT = _NPAD // _NSUB
_OCH = 128
_DEPTH = 2


def _mm_body(x_ref, w_ref, o_ref):
    x = x_ref[...]
    for i in range(_NSUP):
        o_ref[i] = jnp.dot(x, w_ref[i], preferred_element_type=jnp.float32)


def _project(x_all, weights):
    m = x_all.shape[0]
    mb = 1000
    return pl.pallas_call(
        _mm_body,
        grid=(m // mb,),
        in_specs=[
            pl.BlockSpec((mb, _D_IN), lambda r: (r, 0)),
            pl.BlockSpec((_NSUP, _D_IN, _D_PER), lambda r: (0, 0, 0)),
        ],
        out_specs=pl.BlockSpec((_NSUP, mb, _D_PER), lambda r: (0, r, 0)),
        out_shape=jax.ShapeDtypeStruct((_NSUP, m, _D_PER), jnp.float32),
    )(x_all, weights)


def _sc_body(tab, src_hbm, dst_hbm, vals_hbm, out_hbm,
             src2, dst2, vals2, rb0, rb1, rb2, sb0, sb1, sb2, acc, sem_g, sem_s):
    c = lax.axis_index("c")
    s = lax.axis_index("s")
    nd = _DEPTH
    rbl = (rb0, rb1, rb2)[:nd]
    sbl = (sb0, sb1, sb2)[:nd]

    for tt in range(4):
        t = c * 4 + tt

        pltpu.sync_copy(src_hbm.at[t, s], src2)
        pltpu.sync_copy(dst_hbm.at[t, s], dst2)
        pltpu.sync_copy(vals_hbm.at[t, s], vals2)

        def _zb(r, carry):
            for k in range(4):
                sb0[r, pl.ds(k * 16, 16)] = jnp.zeros((16,), jnp.float32)
            return carry
        lax.fori_loop(0, _OCH, _zb, 0)
        for k in range(_RPT // _OCH):
            pltpu.sync_copy(sb0.at[pl.ds(0, _OCH)],
                            acc.at[pl.ds(s * _RPT + k * _OCH, _OCH)])
        plsc.subcore_barrier()

        def _scale(j, rb, sb):
            def _sg(g, c2):
                vv = vals2[j, pl.ds(g * 16, 16)]
                for lane in range(16):
                    e = g * 16 + lane
                    v = vv[lane]
                    for k in range(4):
                        sl = pl.ds(k * 16, 16)
                        sb[e, sl] = rb[e, sl] * v
                return c2
            lax.fori_loop(0, _CH // 16, _sg, 0)

        for j0 in range(nd):
            pltpu.async_copy(tab.at[src2.at[j0]], rbl[j0], sem_g)

        def _step(st, carry):
            for k in range(nd):
                j = nd * st + k
                rb, sb = rbl[k], sbl[k]
                pltpu.make_async_copy(tab.at[src2.at[j]], rb, sem_g).wait()

                @pl.when(st > 0)
                def _drain():
                    pltpu.make_async_copy(sb, acc.at[dst2.at[j]],
                                          sem_s).wait()

                _scale(j, rb, sb)

                @pl.when(st < _NCHUNK // nd - 1)
                def _next():
                    pltpu.async_copy(tab.at[src2.at[j + nd]], rb, sem_g)

                pltpu.async_copy(sb, acc.at[dst2.at[j]], sem_s,
                                 add=True)
            return carry
        lax.fori_loop(0, _NCHUNK // nd, _step, 0)
        for j0 in range(nd):
            pltpu.make_async_copy(sbl[j0], acc.at[dst2.at[j0]],
                                  sem_s).wait()
        plsc.subcore_barrier()

        for k in range(_RPT // _OCH):
            r0 = s * _RPT + k * _OCH
            pltpu.sync_copy(acc.at[pl.ds(r0, _OCH)], rb0.at[pl.ds(0, _OCH)])

            def _relu(r, c3):
                for q in range(4):
                    sl = pl.ds(q * 16, 16)
                    rb0[r, sl] = jnp.maximum(rb0[r, sl], 0.0)
                return c3
            lax.fori_loop(0, _OCH, _relu, 0)
            pltpu.sync_copy(rb0.at[pl.ds(0, _OCH)],
                            out_hbm.at[t, pl.ds(r0, _OCH)])


def _sc_scatter(tab_flat, src_all, dst_all, vals_all):
    mesh = plsc.VectorSubcoreMesh(core_axis_name="c", subcore_axis_name="s")
    f = pl.kernel(
        _sc_body,
        out_type=jax.ShapeDtypeStruct((2 * _NSUP, _NPAD, _D_PER), jnp.float32),
        mesh=mesh,
        scratch_types=[
            pltpu.VMEM((_NCHUNK, _CH), jnp.int32),
            pltpu.VMEM((_NCHUNK, _CH), jnp.int32),
            pltpu.VMEM((_NCHUNK, _CH), jnp.float32),
            pltpu.VMEM((_CH, _D_PER), jnp.float32),
            pltpu.VMEM((_CH, _D_PER), jnp.float32),
            pltpu.VMEM((_CH, _D_PER), jnp.float32),
            pltpu.VMEM((_CH, _D_PER), jnp.float32),
            pltpu.VMEM((_CH, _D_PER), jnp.float32),
            pltpu.VMEM((_CH, _D_PER), jnp.float32),
            pltpu.VMEM_SHARED((_NPAD, _D_PER), jnp.float32),
            pltpu.SemaphoreType.DMA,
            pltpu.SemaphoreType.DMA,
        ],
        compiler_params=pltpu.CompilerParams(use_tc_tiling_on_sc=False),
    )
    return f(tab_flat, src_all, dst_all, vals_all)


def kernel(x_u, x_v, edge_index, edge_values, weights_u):
    ei = edge_index.astype(jnp.int32)
    rows, cols = ei[:, 0], ei[:, 1]
    vals = edge_values.astype(jnp.float32)

    x_all = jnp.concatenate([x_u, x_v], axis=0)
    tab = _project(x_all, weights_u)
    tab_flat = tab.reshape(_NSUP * 2 * _N, _D_PER)

    sup = jnp.arange(_NSUP, dtype=jnp.int32)[:, None]
    src_all = jnp.stack([cols + sup * (2 * _N) + _N,
                         rows + sup * (2 * _N)], axis=1).reshape(8, _N_EDGES)
    dst_all = jnp.stack([rows, cols], axis=1).reshape(8, _N_EDGES)
    vals_all = jnp.stack([vals, vals], axis=1).reshape(8, _N_EDGES)

    pad = _EPAD - _N_EDGES
    src_all = jnp.pad(src_all, ((0, 0), (0, pad)))
    dst_all = jnp.pad(dst_all, ((0, 0), (0, pad)))
    vals_all = jnp.pad(vals_all, ((0, 0), (0, pad)))
    shape4 = (8, _NSUB, _NCHUNK, _CH)

    out8 = _sc_scatter(tab_flat,
                       src_all.reshape(shape4),
                       dst_all.reshape(shape4),
                       vals_all.reshape(shape4))[:, :_N]

    z_u = out8[0::2].transpose(1, 0, 2).reshape(_N, _NSUP * _D_PER)
    z_v = out8[1::2].transpose(1, 0, 2).reshape(_N, _NSUP * _D_PER)
    return z_u, z_v


# speedup vs baseline: 1.4991x; 1.1593x over previous
import functools

import jax
import jax.numpy as jnp
from jax import lax
from jax.experimental import pallas as pl
from jax.experimental.pallas import tpu as pltpu
from jax.experimental.pallas import tpu_sc as plsc

_N = 10000
_D_IN = 256
_NSUP = 4
_D_PER = 64
_N_EDGES = 160000

_NC = 2
_NSUB = 16
_CH = 128
_NCHUNK = 80
_EPT = _CH * _NCHUNK
_EPAD = _EPT * _NSUB
_NPAD = 10240
_RPT = _NPAD // _NSUB
_OCH = 128
_COLMAP = [32 * g + (i // 2) + 16 * (i % 2) for g in range(2) for i in range(32)]
_DEPTH = 2


def _mm_body(x_ref, w_ref, o_ref):
    x = x_ref[...]
    for i in range(_NSUP):
        o_ref[i] = jnp.dot(x, w_ref[i], preferred_element_type=jnp.float32).astype(jnp.bfloat16)


def _project(x_all, weights):
    m = x_all.shape[0]
    mb = 2000
    return pl.pallas_call(
        _mm_body,
        grid=(m // mb,),
        in_specs=[
            pl.BlockSpec((mb, _D_IN), lambda r: (r, 0)),
            pl.BlockSpec((_NSUP, _D_IN, _D_PER), lambda r: (0, 0, 0)),
        ],
        out_specs=pl.BlockSpec((_NSUP, mb, _D_PER), lambda r: (0, r, 0)),
        out_shape=jax.ShapeDtypeStruct((_NSUP, m, _D_PER), jnp.bfloat16),
    )(x_all, weights)


def _sc_body(tab, src_hbm, dst_hbm, vals_hbm, out_hbm,
             src2, dst2, vals2, rb0, rb1, rb2, sb0, sb1, sb2, acc, sem_g, sem_s):
    c = lax.axis_index("c")
    s = lax.axis_index("s")
    nd = _DEPTH
    rbl = (rb0, rb1, rb2)[:nd]
    sbl = (sb0, sb1, sb2)[:nd]

    for tt in range(4):
        t = c * 4 + tt

        pltpu.sync_copy(src_hbm.at[t, s], src2)
        pltpu.sync_copy(dst_hbm.at[t, s], dst2)
        pltpu.sync_copy(vals_hbm.at[t, s], vals2)

        def _zb(r, carry):
            for k in range(4):
                sb0[r, pl.ds(k * 16, 16)] = jnp.zeros((16,), jnp.float32)
            return carry
        lax.fori_loop(0, _OCH, _zb, 0)
        for k in range(_RPT // _OCH):
            pltpu.sync_copy(sb0.at[pl.ds(0, _OCH)],
                            acc.at[pl.ds(s * _RPT + k * _OCH, _OCH)])
        plsc.subcore_barrier()

        def _scale(j, rb, sb):
            def _sg(g, c2):
                vv = vals2[j, pl.ds(g * 16, 16)]
                for lane in range(16):
                    e = g * 16 + lane
                    v = vv[lane]
                    for h in range(2):
                        ab = rb[e, pl.ds(32 * h, 32)]
                        a, b = plsc.unpack(ab, format=plsc.PackFormat.INTERLEAVED)
                        sb[e, pl.ds(32 * h, 16)] = a * v
                        sb[e, pl.ds(32 * h + 16, 16)] = b * v
                return c2
            lax.fori_loop(0, _CH // 16, _sg, 0)

        for j0 in range(nd):
            pltpu.async_copy(tab.at[src2.at[j0]], rbl[j0], sem_g)

        def _step(st, carry):
            for k in range(nd):
                j = nd * st + k
                rb, sb = rbl[k], sbl[k]
                pltpu.make_async_copy(tab.at[src2.at[j]], rb, sem_g).wait()

                @pl.when(st > 0)
                def _drain():
                    pltpu.make_async_copy(sb, acc.at[dst2.at[j]],
                                          sem_s).wait()

                _scale(j, rb, sb)

                @pl.when(st < _NCHUNK // nd - 1)
                def _next():
                    pltpu.async_copy(tab.at[src2.at[j + nd]], rb, sem_g)

                pltpu.async_copy(sb, acc.at[dst2.at[j]], sem_s,
                                 add=True)
            return carry
        lax.fori_loop(0, _NCHUNK // nd, _step, 0)
        for j0 in range(nd):
            pltpu.make_async_copy(sbl[j0], acc.at[dst2.at[j0]],
                                  sem_s).wait()
        plsc.subcore_barrier()

        for k in range(_RPT // _OCH):
            r0 = s * _RPT + k * _OCH
            pltpu.sync_copy(acc.at[pl.ds(r0, _OCH)], sb0.at[pl.ds(0, _OCH)])

            def _relu(r, c3):
                for q in range(4):
                    sl = pl.ds(q * 16, 16)
                    sb0[r, sl] = jnp.maximum(sb0[r, sl], 0.0)
                return c3
            lax.fori_loop(0, _OCH, _relu, 0)
            pltpu.sync_copy(sb0.at[pl.ds(0, _OCH)],
                            out_hbm.at[t, pl.ds(r0, _OCH)])


def _sc_scatter(tab_flat, src_all, dst_all, vals_all):
    mesh = plsc.VectorSubcoreMesh(core_axis_name="c", subcore_axis_name="s")
    f = pl.kernel(
        _sc_body,
        out_type=jax.ShapeDtypeStruct((2 * _NSUP, _NPAD, _D_PER), jnp.float32),
        mesh=mesh,
        scratch_types=[
            pltpu.VMEM((_NCHUNK, _CH), jnp.int32),
            pltpu.VMEM((_NCHUNK, _CH), jnp.int32),
            pltpu.VMEM((_NCHUNK, _CH), jnp.float32),
            pltpu.VMEM((_CH, _D_PER), jnp.bfloat16),
            pltpu.VMEM((_CH, _D_PER), jnp.bfloat16),
            pltpu.VMEM((_CH, _D_PER), jnp.bfloat16),
            pltpu.VMEM((_CH, _D_PER), jnp.float32),
            pltpu.VMEM((_CH, _D_PER), jnp.float32),
            pltpu.VMEM((_CH, _D_PER), jnp.float32),
            pltpu.VMEM_SHARED((_NPAD, _D_PER), jnp.float32),
            pltpu.SemaphoreType.DMA,
            pltpu.SemaphoreType.DMA,
        ],
        compiler_params=pltpu.CompilerParams(use_tc_tiling_on_sc=False,
                                             needs_layout_passes=False),
    )
    return f(tab_flat, src_all, dst_all, vals_all)


def kernel(x_u, x_v, edge_index, edge_values, weights_u):
    ei = edge_index.astype(jnp.int32)
    rows, cols = ei[:, 0], ei[:, 1]
    vals = edge_values.astype(jnp.float32)

    x_all = jnp.concatenate([x_u, x_v], axis=0)
    w_perm = weights_u[:, :, jnp.array(_COLMAP, dtype=jnp.int32)]
    tab = _project(x_all, w_perm)
    tab_flat = tab.reshape(_NSUP * 2 * _N, _D_PER)

    sup = jnp.arange(_NSUP, dtype=jnp.int32)[:, None]
    src_all = jnp.stack([cols + sup * (2 * _N) + _N,
                         rows + sup * (2 * _N)], axis=1).reshape(8, _N_EDGES)
    dst_all = jnp.stack([rows, cols], axis=1).reshape(8, _N_EDGES)
    vals_all = jnp.stack([vals, vals], axis=1).reshape(8, _N_EDGES)

    pad = _EPAD - _N_EDGES
    src_all = jnp.pad(src_all, ((0, 0), (0, pad)))
    dst_all = jnp.pad(dst_all, ((0, 0), (0, pad)))
    vals_all = jnp.pad(vals_all, ((0, 0), (0, pad)))
    shape4 = (8, _NSUB, _NCHUNK, _CH)

    out8 = _sc_scatter(tab_flat,
                       src_all.reshape(shape4),
                       dst_all.reshape(shape4),
                       vals_all.reshape(shape4))[:, :_N]

    z_u = out8[0::2].transpose(1, 0, 2).reshape(_N, _NSUP * _D_PER)
    z_v = out8[1::2].transpose(1, 0, 2).reshape(_N, _NSUP * _D_PER)
    return z_u, z_v


# speedup vs baseline: 1.5021x; 1.0020x over previous
import functools

import jax
import jax.numpy as jnp
from jax import lax
from jax.experimental import pallas as pl
from jax.experimental.pallas import tpu as pltpu
from jax.experimental.pallas import tpu_sc as plsc

_N = 10000
_D_IN = 256
_NSUP = 4
_D_PER = 64
_N_EDGES = 160000

_NC = 2
_NSUB = 16
_CH = 256
_NCHUNK = 40
_EPT = _CH * _NCHUNK
_EPAD = _EPT * _NSUB
_NPAD = 10240
_RPT = _NPAD // _NSUB
_OCH = 128
_COLMAP = [32 * g + (i // 2) + 16 * (i % 2) for g in range(2) for i in range(32)]
_DEPTH = 2


def _mm_body(x_ref, w_ref, o_ref):
    x = x_ref[...]
    for i in range(_NSUP):
        o_ref[i] = jnp.dot(x, w_ref[i], preferred_element_type=jnp.float32).astype(jnp.bfloat16)


def _project(x_all, weights):
    m = x_all.shape[0]
    mb = 2000
    return pl.pallas_call(
        _mm_body,
        grid=(m // mb,),
        in_specs=[
            pl.BlockSpec((mb, _D_IN), lambda r: (r, 0)),
            pl.BlockSpec((_NSUP, _D_IN, _D_PER), lambda r: (0, 0, 0)),
        ],
        out_specs=pl.BlockSpec((_NSUP, mb, _D_PER), lambda r: (0, r, 0)),
        out_shape=jax.ShapeDtypeStruct((_NSUP, m, _D_PER), jnp.bfloat16),
    )(x_all, weights)


def _sc_body(tab, src_hbm, dst_hbm, vals_hbm, out_hbm,
             src2, dst2, vals2, rb0, rb1, rb2, sb0, sb1, sb2, acc, sem_g, sem_s):
    c = lax.axis_index("c")
    s = lax.axis_index("s")
    nd = _DEPTH
    rbl = (rb0, rb1, rb2)[:nd]
    sbl = (sb0, sb1, sb2)[:nd]

    for tt in range(4):
        t = c * 4 + tt

        pltpu.sync_copy(src_hbm.at[t, s], src2)
        pltpu.sync_copy(dst_hbm.at[t, s], dst2)
        pltpu.sync_copy(vals_hbm.at[t, s], vals2)

        def _zb(r, carry):
            for k in range(4):
                sb0[r, pl.ds(k * 16, 16)] = jnp.zeros((16,), jnp.float32)
            return carry
        lax.fori_loop(0, _OCH, _zb, 0)
        for k in range(_RPT // _OCH):
            pltpu.sync_copy(sb0.at[pl.ds(0, _OCH)],
                            acc.at[pl.ds(s * _RPT + k * _OCH, _OCH)])
        plsc.subcore_barrier()

        def _scale(j, rb, sb):
            def _sg(g, c2):
                vv = vals2[j, pl.ds(g * 16, 16)]
                for lane in range(16):
                    e = g * 16 + lane
                    v = vv[lane]
                    for h in range(2):
                        ab = rb[e, pl.ds(32 * h, 32)]
                        a, b = plsc.unpack(ab, format=plsc.PackFormat.INTERLEAVED)
                        sb[e, pl.ds(32 * h, 16)] = a * v
                        sb[e, pl.ds(32 * h + 16, 16)] = b * v
                return c2
            lax.fori_loop(0, _CH // 16, _sg, 0)

        for j0 in range(nd):
            pltpu.async_copy(tab.at[src2.at[j0]], rbl[j0], sem_g)

        def _step(st, carry):
            for k in range(nd):
                j = nd * st + k
                rb, sb = rbl[k], sbl[k]
                pltpu.make_async_copy(tab.at[src2.at[j]], rb, sem_g).wait()

                @pl.when(st > 0)
                def _drain():
                    pltpu.make_async_copy(sb, acc.at[dst2.at[j]],
                                          sem_s).wait()

                _scale(j, rb, sb)

                @pl.when(st < _NCHUNK // nd - 1)
                def _next():
                    pltpu.async_copy(tab.at[src2.at[j + nd]], rb, sem_g)

                pltpu.async_copy(sb, acc.at[dst2.at[j]], sem_s,
                                 add=True)
            return carry
        lax.fori_loop(0, _NCHUNK // nd, _step, 0)
        for j0 in range(nd):
            pltpu.make_async_copy(sbl[j0], acc.at[dst2.at[j0]],
                                  sem_s).wait()
        plsc.subcore_barrier()

        for k in range(_RPT // _OCH):
            r0 = s * _RPT + k * _OCH
            pltpu.sync_copy(acc.at[pl.ds(r0, _OCH)], sb0.at[pl.ds(0, _OCH)])

            def _relu(r, c3):
                for q in range(4):
                    sl = pl.ds(q * 16, 16)
                    sb0[r, sl] = jnp.maximum(sb0[r, sl], 0.0)
                return c3
            lax.fori_loop(0, _OCH, _relu, 0)
            pltpu.sync_copy(sb0.at[pl.ds(0, _OCH)],
                            out_hbm.at[t, pl.ds(r0, _OCH)])


def _sc_scatter(tab_flat, src_all, dst_all, vals_all):
    mesh = plsc.VectorSubcoreMesh(core_axis_name="c", subcore_axis_name="s")
    f = pl.kernel(
        _sc_body,
        out_type=jax.ShapeDtypeStruct((2 * _NSUP, _NPAD, _D_PER), jnp.float32),
        mesh=mesh,
        scratch_types=[
            pltpu.VMEM((_NCHUNK, _CH), jnp.int32),
            pltpu.VMEM((_NCHUNK, _CH), jnp.int32),
            pltpu.VMEM((_NCHUNK, _CH), jnp.float32),
            pltpu.VMEM((_CH, _D_PER), jnp.bfloat16),
            pltpu.VMEM((_CH, _D_PER), jnp.bfloat16),
            pltpu.VMEM((_CH, _D_PER), jnp.bfloat16),
            pltpu.VMEM((_CH, _D_PER), jnp.float32),
            pltpu.VMEM((_CH, _D_PER), jnp.float32),
            pltpu.VMEM((_CH, _D_PER), jnp.float32),
            pltpu.VMEM_SHARED((_NPAD, _D_PER), jnp.float32),
            pltpu.SemaphoreType.DMA,
            pltpu.SemaphoreType.DMA,
        ],
        compiler_params=pltpu.CompilerParams(use_tc_tiling_on_sc=False,
                                             needs_layout_passes=False),
    )
    return f(tab_flat, src_all, dst_all, vals_all)


def kernel(x_u, x_v, edge_index, edge_values, weights_u):
    ei = edge_index.astype(jnp.int32)
    rows, cols = ei[:, 0], ei[:, 1]
    vals = edge_values.astype(jnp.float32)

    x_all = jnp.concatenate([x_u, x_v], axis=0)
    w_perm = weights_u[:, :, jnp.array(_COLMAP, dtype=jnp.int32)]
    tab = _project(x_all, w_perm)
    tab_flat = tab.reshape(_NSUP * 2 * _N, _D_PER)

    sup = jnp.arange(_NSUP, dtype=jnp.int32)[:, None]
    src_all = jnp.stack([cols + sup * (2 * _N) + _N,
                         rows + sup * (2 * _N)], axis=1).reshape(8, _N_EDGES)
    dst_all = jnp.stack([rows, cols], axis=1).reshape(8, _N_EDGES)
    vals_all = jnp.stack([vals, vals], axis=1).reshape(8, _N_EDGES)

    pad = _EPAD - _N_EDGES
    src_all = jnp.pad(src_all, ((0, 0), (0, pad)))
    dst_all = jnp.pad(dst_all, ((0, 0), (0, pad)))
    vals_all = jnp.pad(vals_all, ((0, 0), (0, pad)))
    shape4 = (8, _NSUB, _NCHUNK, _CH)

    out8 = _sc_scatter(tab_flat,
                       src_all.reshape(shape4),
                       dst_all.reshape(shape4),
                       vals_all.reshape(shape4))[:, :_N]

    z_u = out8[0::2].transpose(1, 0, 2).reshape(_N, _NSUP * _D_PER)
    z_v = out8[1::2].transpose(1, 0, 2).reshape(_N, _NSUP * _D_PER)
    return z_u, z_v


# speedup vs baseline: 1.6971x; 1.1298x over previous
import functools

import jax
import jax.numpy as jnp
from jax import lax
from jax.experimental import pallas as pl
from jax.experimental.pallas import tpu as pltpu
from jax.experimental.pallas import tpu_sc as plsc

_N = 10000
_D_IN = 256
_NSUP = 4
_D_PER = 64
_N_EDGES = 160000

_NC = 2
_NSUB = 16
_CH = 256
_NCHUNK = 40
_EPT = _CH * _NCHUNK
_EPAD = _EPT * _NSUB
_NPAD = 10240
_RPT = _NPAD // _NSUB
_OCH = 128
_COLMAP = [32 * g + (i // 2) + 16 * (i % 2) for g in range(2) for i in range(32)]
_DEPTH = 2


def _mm_body(x_ref, w_ref, o_ref):
    x = x_ref[...]
    for i in range(_NSUP):
        o_ref[i] = jnp.dot(x, w_ref[i], preferred_element_type=jnp.float32).astype(jnp.bfloat16)


def _project(x_all, weights):
    m = x_all.shape[0]
    mb = 2000
    return pl.pallas_call(
        _mm_body,
        grid=(m // mb,),
        in_specs=[
            pl.BlockSpec((mb, _D_IN), lambda r: (r, 0)),
            pl.BlockSpec((_NSUP, _D_IN, _D_PER), lambda r: (0, 0, 0)),
        ],
        out_specs=pl.BlockSpec((_NSUP, mb, _D_PER), lambda r: (0, r, 0)),
        out_shape=jax.ShapeDtypeStruct((_NSUP, m, _D_PER), jnp.bfloat16),
    )(x_all, weights)


def _sc_body(tab, src_hbm, dst_hbm, vals_hbm, out_u, out_v,
             src2, dst2, vals2, rb0, rb1, rb2, sb0, sb1, sb2, acc, sem_g, sem_s):
    c = lax.axis_index("c")
    s = lax.axis_index("s")
    nd = _DEPTH
    rbl = (rb0, rb1, rb2)[:nd]
    sbl = (sb0, sb1, sb2)[:nd]

    for tt in range(4):
        t = c * 4 + tt

        pltpu.sync_copy(src_hbm.at[t, s], src2)
        pltpu.sync_copy(dst_hbm.at[t, s], dst2)
        pltpu.sync_copy(vals_hbm.at[t, s], vals2)

        def _zb(r, carry):
            for k in range(4):
                sb0[r, pl.ds(k * 16, 16)] = jnp.zeros((16,), jnp.float32)
            return carry
        lax.fori_loop(0, _OCH, _zb, 0)
        for k in range(_RPT // _OCH):
            pltpu.sync_copy(sb0.at[pl.ds(0, _OCH)],
                            acc.at[pl.ds(s * _RPT + k * _OCH, _OCH)])
        plsc.subcore_barrier()

        def _scale(j, rb, sb):
            def _sg(g, c2):
                vv = vals2[j, pl.ds(g * 16, 16)]
                for lane in range(16):
                    e = g * 16 + lane
                    v = vv[lane]
                    for h in range(2):
                        ab = rb[e, pl.ds(32 * h, 32)]
                        a, b = plsc.unpack(ab, format=plsc.PackFormat.INTERLEAVED)
                        sb[e, pl.ds(32 * h, 16)] = a * v
                        sb[e, pl.ds(32 * h + 16, 16)] = b * v
                return c2
            lax.fori_loop(0, _CH // 16, _sg, 0)

        for j0 in range(nd):
            pltpu.async_copy(tab.at[src2.at[j0]], rbl[j0], sem_g)

        def _step(st, carry):
            for k in range(nd):
                j = nd * st + k
                rb, sb = rbl[k], sbl[k]
                pltpu.make_async_copy(tab.at[src2.at[j]], rb, sem_g).wait()

                @pl.when(st > 0)
                def _drain():
                    pltpu.make_async_copy(sb, acc.at[dst2.at[j]],
                                          sem_s).wait()

                _scale(j, rb, sb)

                @pl.when(st < _NCHUNK // nd - 1)
                def _next():
                    pltpu.async_copy(tab.at[src2.at[j + nd]], rb, sem_g)

                pltpu.async_copy(sb, acc.at[dst2.at[j]], sem_s,
                                 add=True)
            return carry
        lax.fori_loop(0, _NCHUNK // nd, _step, 0)
        for j0 in range(nd):
            pltpu.make_async_copy(sbl[j0], acc.at[dst2.at[j0]],
                                  sem_s).wait()
        plsc.subcore_barrier()

        out_ref = out_u if tt % 2 == 0 else out_v
        col0 = 64 * (2 * c + tt // 2)
        for k in range(_RPT // _OCH):
            r0 = s * _RPT + k * _OCH
            pltpu.sync_copy(acc.at[pl.ds(r0, _OCH)], sb0.at[pl.ds(0, _OCH)])

            def _relu(r, c3):
                for q in range(4):
                    sl = pl.ds(q * 16, 16)
                    sb0[r, sl] = jnp.maximum(sb0[r, sl], 0.0)
                return c3
            lax.fori_loop(0, _OCH, _relu, 0)

            @pl.when(r0 + _OCH <= _N)
            def _full():
                pltpu.sync_copy(sb0.at[pl.ds(0, _OCH)],
                                out_ref.at[pl.ds(r0, _OCH), pl.ds(col0, _D_PER)])

            @pl.when(r0 == _N - 16)
            def _tail():
                pltpu.sync_copy(sb0.at[pl.ds(0, 16)],
                                out_ref.at[pl.ds(r0, 16), pl.ds(col0, _D_PER)])


def _sc_scatter(tab_flat, src_all, dst_all, vals_all):
    mesh = plsc.VectorSubcoreMesh(core_axis_name="c", subcore_axis_name="s")
    f = pl.kernel(
        _sc_body,
        out_type=(jax.ShapeDtypeStruct((_N, _NSUP * _D_PER), jnp.float32),
                  jax.ShapeDtypeStruct((_N, _NSUP * _D_PER), jnp.float32)),
        mesh=mesh,
        scratch_types=[
            pltpu.VMEM((_NCHUNK, _CH), jnp.int32),
            pltpu.VMEM((_NCHUNK, _CH), jnp.int32),
            pltpu.VMEM((_NCHUNK, _CH), jnp.float32),
            pltpu.VMEM((_CH, _D_PER), jnp.bfloat16),
            pltpu.VMEM((_CH, _D_PER), jnp.bfloat16),
            pltpu.VMEM((_CH, _D_PER), jnp.bfloat16),
            pltpu.VMEM((_CH, _D_PER), jnp.float32),
            pltpu.VMEM((_CH, _D_PER), jnp.float32),
            pltpu.VMEM((_CH, _D_PER), jnp.float32),
            pltpu.VMEM_SHARED((_NPAD, _D_PER), jnp.float32),
            pltpu.SemaphoreType.DMA,
            pltpu.SemaphoreType.DMA,
        ],
        compiler_params=pltpu.CompilerParams(use_tc_tiling_on_sc=False,
                                             needs_layout_passes=False),
    )
    return f(tab_flat, src_all, dst_all, vals_all)


def kernel(x_u, x_v, edge_index, edge_values, weights_u):
    ei = edge_index.astype(jnp.int32)
    rows, cols = ei[:, 0], ei[:, 1]
    vals = edge_values.astype(jnp.float32)

    x_all = jnp.concatenate([x_u, x_v], axis=0)
    w_perm = weights_u[:, :, jnp.array(_COLMAP, dtype=jnp.int32)]
    tab = _project(x_all, w_perm)
    tab_flat = tab.reshape(_NSUP * 2 * _N, _D_PER)

    sup = jnp.arange(_NSUP, dtype=jnp.int32)[:, None]
    src_all = jnp.stack([cols + sup * (2 * _N) + _N,
                         rows + sup * (2 * _N)], axis=1).reshape(8, _N_EDGES)
    dst_all = jnp.stack([rows, cols], axis=1).reshape(8, _N_EDGES)
    vals_all = jnp.stack([vals, vals], axis=1).reshape(8, _N_EDGES)

    pad = _EPAD - _N_EDGES
    src_all = jnp.pad(src_all, ((0, 0), (0, pad)))
    dst_all = jnp.pad(dst_all, ((0, 0), (0, pad)))
    vals_all = jnp.pad(vals_all, ((0, 0), (0, pad)))
    shape4 = (8, _NSUB, _NCHUNK, _CH)

    z_u, z_v = _sc_scatter(tab_flat,
                           src_all.reshape(shape4),
                           dst_all.reshape(shape4),
                           vals_all.reshape(shape4))
    return z_u, z_v
